# baseline (speedup 1.0000x reference)
import jax
import jax.numpy as jnp
from jax import lax
from jax.experimental import pallas as pl
from jax.experimental.pallas import tpu as pltpu
from jax.experimental.pallas import tpu_sc as plsc

N = 10000
E = 320000
D = 128
NL = 6
NSUB = 16
NCORE = 2
CHUNK = 128
SUP = 8
HSPLIT = 2
HS = CHUNK // HSPLIT
NSUP = 20
KC_G = NSUP * SUP
EPAD_G = NSUB * KC_G * CHUNK
ET = E // NSUB
KC_S = ET // CHUNK
TAIL = ET - KC_S * CHUNK
KC_SP = KC_S + 1
NPAD = 10112
DUMMY = N
RZ = NPAD // NSUB


def _sc_gather_body(x_hbm, srci_hbm, dsti_hbm, zeros_hbm, g_hbm,
                    src_v, dst_v, rows0, rows1, acc_sh, sem0, sem1):
    cid = lax.axis_index("c")
    sid = lax.axis_index("s")
    pltpu.sync_copy(zeros_hbm.at[pl.ds(sid * RZ, RZ)],
                    acc_sh.at[pl.ds(sid * RZ, RZ)])
    plsc.subcore_barrier()

    def gather_split(j, src_v_, rows, sem):
        return [pltpu.async_copy(
                    x_hbm.at[src_v_.at[j, pl.ds(h * HS, HS)]],
                    rows.at[pl.ds(h * HS, HS)], sem)
                for h in range(HSPLIT)]

    def body(o, carry):
        pltpu.sync_copy(srci_hbm.at[cid, sid, pl.ds(o * SUP, SUP)], src_v)
        pltpu.sync_copy(dsti_hbm.at[cid, sid, pl.ds(o * SUP, SUP)], dst_v)
        for m in range(SUP // 2):
            j0 = 2 * m
            j1 = j0 + 1
            cps0 = gather_split(j0, src_v, rows0, sem0)
            cps1 = gather_split(j1, src_v, rows1, sem1)
            for cp in cps0:
                cp.wait()
            pltpu.sync_copy(rows0, acc_sh.at[dst_v.at[j0]], add=True)
            for cp in cps1:
                cp.wait()
            pltpu.sync_copy(rows1, acc_sh.at[dst_v.at[j1]], add=True)
        return carry

    lax.fori_loop(0, NSUP, body, 0)
    plsc.subcore_barrier()
    pltpu.sync_copy(acc_sh.at[pl.ds(sid * RZ, RZ)],
                    g_hbm.at[cid, pl.ds(sid * RZ, RZ)])


_sc_gather = pl.kernel(
    _sc_gather_body,
    out_type=jax.ShapeDtypeStruct((NCORE, NPAD, D), jnp.float32),
    mesh=plsc.VectorSubcoreMesh(core_axis_name="c", subcore_axis_name="s"),
    scratch_types=[
        pltpu.VMEM((SUP, CHUNK), jnp.int32),
        pltpu.VMEM((SUP, CHUNK), jnp.int32),
        pltpu.VMEM((CHUNK, D), jnp.float32),
        pltpu.VMEM((CHUNK, D), jnp.float32),
        pltpu.VMEM_SHARED((NPAD, D), jnp.float32),
        pltpu.SemaphoreType.DMA,
        pltpu.SemaphoreType.DMA,
    ],
)


def _sc_segsum_body(r_hbm, ir_hbm, dsti_hbm, zeros_hbm, s_hbm,
                    dst_v, rows0, acc_sh, sem0):
    cid = lax.axis_index("c")
    sid = lax.axis_index("s")
    pltpu.sync_copy(zeros_hbm.at[pl.ds(sid * RZ, RZ)],
                    acc_sh.at[pl.ds(sid * RZ, RZ)])
    pltpu.sync_copy(dsti_hbm.at[cid, sid], dst_v)
    plsc.subcore_barrier()
    base = sid * ET

    def run(src_ref):
        def body(j, carry):
            pltpu.sync_copy(src_ref.at[pl.ds(base + j * CHUNK, CHUNK)], rows0)
            pltpu.sync_copy(rows0, acc_sh.at[dst_v.at[j]], add=True)
            return carry

        lax.fori_loop(0, KC_S, body, 0)
        pltpu.sync_copy(zeros_hbm.at[pl.ds(0, CHUNK - TAIL)],
                        rows0.at[pl.ds(TAIL, CHUNK - TAIL)])
        pltpu.sync_copy(src_ref.at[pl.ds(base + KC_S * CHUNK, TAIL)],
                        rows0.at[pl.ds(0, TAIL)])
        pltpu.sync_copy(rows0, acc_sh.at[dst_v.at[KC_S]], add=True)

    @pl.when(cid == 0)
    def _():
        run(r_hbm)

    @pl.when(cid == 1)
    def _():
        run(ir_hbm)

    plsc.subcore_barrier()
    pltpu.sync_copy(acc_sh.at[pl.ds(sid * RZ, RZ)],
                    s_hbm.at[cid, pl.ds(sid * RZ, RZ)])


_sc_segsum = pl.kernel(
    _sc_segsum_body,
    out_type=jax.ShapeDtypeStruct((NCORE, NPAD, D), jnp.float32),
    mesh=plsc.VectorSubcoreMesh(core_axis_name="c", subcore_axis_name="s"),
    scratch_types=[
        pltpu.VMEM((KC_SP, CHUNK), jnp.int32),
        pltpu.VMEM((CHUNK, D), jnp.float32),
        pltpu.VMEM_SHARED((NPAD, D), jnp.float32),
        pltpu.SemaphoreType.DMA,
    ],
)


def _wprep_body(wrel_ref, win_ref, wout_ref, ab_ref, c6_ref):
    ii = lax.broadcasted_iota(jnp.int32, (D, D), 0)
    jj = lax.broadcasted_iota(jnp.int32, (D, D), 1)
    c = (ii == jj).astype(jnp.float32)
    for l in range(NL):
        ab_ref[0, l] = jnp.dot(c, win_ref[l], preferred_element_type=jnp.float32)
        ab_ref[1, l] = jnp.dot(c, wout_ref[l], preferred_element_type=jnp.float32)
        c = jnp.dot(c, wrel_ref[l], preferred_element_type=jnp.float32)
    c6_ref[...] = c


_wprep = pl.pallas_call(
    _wprep_body,
    out_shape=(jax.ShapeDtypeStruct((2, NL, D, D), jnp.float32),
               jax.ShapeDtypeStruct((D, D), jnp.float32)),
)


RB = 2000


def _dense_body(g_ref, s_ref, x_ref, w_ref, b_ref, o_ref):
    acc = jnp.dot(g_ref[0], w_ref[0], preferred_element_type=jnp.float32)
    acc += jnp.dot(g_ref[1], w_ref[1], preferred_element_type=jnp.float32)
    acc += jnp.dot(x_ref[...], w_ref[2], preferred_element_type=jnp.float32)
    acc -= jnp.dot(s_ref[0], w_ref[3], preferred_element_type=jnp.float32)
    acc -= jnp.dot(s_ref[1], w_ref[4], preferred_element_type=jnp.float32)
    o_ref[...] = jnp.tanh(acc / 3.0 + b_ref[...])


_dense = pl.pallas_call(
    _dense_body,
    grid=(N // RB,),
    in_specs=[
        pl.BlockSpec((2, RB, D), lambda i: (0, i, 0)),
        pl.BlockSpec((2, RB, D), lambda i: (0, i, 0)),
        pl.BlockSpec((RB, D), lambda i: (i, 0)),
        pl.BlockSpec((5, D, D), lambda i: (0, 0, 0)),
        pl.BlockSpec((1, D), lambda i: (0, 0)),
    ],
    out_specs=pl.BlockSpec((RB, D), lambda i: (i, 0)),
    out_shape=jax.ShapeDtypeStruct((N, D), jnp.float32),
)

EB = 3200


def _relmm_body(r_ref, c_ref, o_ref):
    o_ref[...] = jnp.dot(r_ref[...], c_ref[...],
                         preferred_element_type=jnp.float32)


_relmm = pl.pallas_call(
    _relmm_body,
    grid=(E // EB,),
    in_specs=[pl.BlockSpec((EB, D), lambda i: (i, 0)),
              pl.BlockSpec((D, D), lambda i: (0, 0))],
    out_specs=pl.BlockSpec((EB, D), lambda i: (i, 0)),
    out_shape=jax.ShapeDtypeStruct((E, D), jnp.float32),
)


def kernel(node_features, edge_features, inverse_edge_features,
           edge_index, inverse_edge_index, params):
    x = node_features
    r0 = edge_features
    ir0 = inverse_edge_features
    ei = edge_index.astype(jnp.int32)
    iei = inverse_edge_index.astype(jnp.int32)
    src, dst = ei[0], ei[1]
    isrc, idst = iei[0], iei[1]

    def pack_g(a, padval):
        return jnp.pad(a, (0, EPAD_G - E),
                       constant_values=padval).reshape(NSUB, KC_G, CHUNK)

    def pack_s(a):
        return jnp.pad(a.reshape(NSUB, ET), ((0, 0), (0, CHUNK - TAIL)),
                       constant_values=DUMMY).reshape(NSUB, KC_SP, CHUNK)

    srcp = jnp.stack([pack_g(src, 0), pack_g(isrc, 0)])
    dstp = jnp.stack([pack_g(dst, DUMMY), pack_g(idst, DUMMY)])
    dsts = jnp.stack([pack_s(dst), pack_s(idst)])
    zeros = jnp.zeros((NPAD, D), jnp.float32)

    wrel = jnp.stack([p['W_rel'] for p in params])
    win = jnp.stack([p['W_in'] for p in params])
    wout = jnp.stack([p['W_out'] for p in params])
    ab, c6 = _wprep(wrel, win, wout)

    s = _sc_segsum(r0, ir0, dsts, zeros)
    x, s = lax.optimization_barrier((x, s))
    for l in range(NL):
        g = _sc_gather(x, srcp, dstp, zeros)
        w5 = jnp.stack([params[l]['W_in'], params[l]['W_out'],
                        params[l]['W_loop'], ab[0, l], ab[1, l]])
        x = _dense(g, s, x, w5, params[l]['b'].reshape(1, D))

    r_out = _relmm(r0, c6)
    ir_out = _relmm(ir0, c6)
    return x, r_out, ir_out


# speedup vs baseline: 1.0112x; 1.0112x over previous
import jax
import jax.numpy as jnp
from jax import lax
from jax.experimental import pallas as pl
from jax.experimental.pallas import tpu as pltpu
from jax.experimental.pallas import tpu_sc as plsc

N = 10000
E = 320000
D = 128
NL = 6
NSUB = 16
NCORE = 2
CHUNK = 128
ET = E // NSUB
KC_S = ET // CHUNK
TAIL = ET - KC_S * CHUNK
KC_SP = KC_S + 1
NACC = 10016
DUMMY = N
SZ = 624
ZTAIL = NACC - NSUB * SZ
XTAIL = N - NSUB * SZ

NH = N // 2
HACC = 5008
HSZ = 304
HTAIL = HACC - NSUB * HSZ
CH = 16
KCG = 1280
IR = KCG * CH // 128
EPAD = NSUB * KCG * CH


def _sc_gather_body(x_hbm, srci_hbm, dst0_hbm, dst1_hbm, zeros_hbm, g_hbm,
                    src_v, dst_v, rows0, rows1, x_sh, acc_sh, sem0, sem1):
    cid = lax.axis_index("c")
    sid = lax.axis_index("s")
    pltpu.sync_copy(x_hbm.at[pl.ds(sid * SZ, SZ)],
                    x_sh.at[pl.ds(sid * SZ, SZ)])

    @pl.when(sid == NSUB - 1)
    def _():
        pltpu.sync_copy(x_hbm.at[pl.ds(NSUB * SZ, XTAIL)],
                        x_sh.at[pl.ds(NSUB * SZ, XTAIL)])

    for h, dsth_hbm in enumerate((dst0_hbm, dst1_hbm)):
        pltpu.sync_copy(zeros_hbm.at[pl.ds(sid * HSZ, HSZ)],
                        acc_sh.at[pl.ds(sid * HSZ, HSZ)])

        @pl.when(sid == NSUB - 1)
        def _():
            pltpu.sync_copy(zeros_hbm.at[pl.ds(NSUB * HSZ, HTAIL)],
                            acc_sh.at[pl.ds(NSUB * HSZ, HTAIL)])

        plsc.subcore_barrier()

        def pair(m, carry):
            @pl.when(lax.rem(m, 32) == 0)
            def _():
                o = lax.div(m, 32) * 8
                pltpu.sync_copy(srci_hbm.at[cid, sid, pl.ds(o, 8)], src_v)
                pltpu.sync_copy(dsth_hbm.at[cid, sid, pl.ds(o, 8)], dst_v)
            c0 = 2 * m
            lr0 = lax.rem(c0, 64) // 8
            of0 = lax.rem(c0, 8) * CH
            lr1 = lax.rem(c0 + 1, 64) // 8
            of1 = lax.rem(c0 + 1, 8) * CH
            si0 = src_v[lr0, pl.ds(of0, CH)]
            di0 = dst_v[lr0, pl.ds(of0, CH)]
            si1 = src_v[lr1, pl.ds(of1, CH)]
            di1 = dst_v[lr1, pl.ds(of1, CH)]
            cp0 = pltpu.async_copy(x_sh.at[si0], rows0, sem0)
            cp1 = pltpu.async_copy(x_sh.at[si1], rows1, sem1)
            cp0.wait()
            pltpu.sync_copy(rows0, acc_sh.at[di0], add=True)
            cp1.wait()
            pltpu.sync_copy(rows1, acc_sh.at[di1], add=True)
            return carry

        lax.fori_loop(0, KCG // 2, pair, 0)
        plsc.subcore_barrier()
        pltpu.sync_copy(acc_sh.at[pl.ds(sid * HSZ, HSZ)],
                        g_hbm.at[cid, h, pl.ds(sid * HSZ, HSZ)])

        @pl.when(sid == NSUB - 1)
        def _():
            pltpu.sync_copy(acc_sh.at[pl.ds(NSUB * HSZ, HTAIL)],
                            g_hbm.at[cid, h, pl.ds(NSUB * HSZ, HTAIL)])


_sc_gather = pl.kernel(
    _sc_gather_body,
    out_type=jax.ShapeDtypeStruct((NCORE, 2, HACC, D), jnp.float32),
    mesh=plsc.VectorSubcoreMesh(core_axis_name="c", subcore_axis_name="s"),
    scratch_types=[
        pltpu.VMEM((8, 128), jnp.int32),
        pltpu.VMEM((8, 128), jnp.int32),
        pltpu.VMEM((CH, D), jnp.float32),
        pltpu.VMEM((CH, D), jnp.float32),
        pltpu.VMEM_SHARED((N, D), jnp.float32),
        pltpu.VMEM_SHARED((HACC, D), jnp.float32),
        pltpu.SemaphoreType.DMA,
        pltpu.SemaphoreType.DMA,
    ],
)


def _sc_segsum_body(r_hbm, ir_hbm, dsti_hbm, zeros_hbm, s_hbm,
                    dst_v, rows0, acc_sh, sem0):
    cid = lax.axis_index("c")
    sid = lax.axis_index("s")
    pltpu.sync_copy(zeros_hbm.at[pl.ds(sid * SZ, SZ)],
                    acc_sh.at[pl.ds(sid * SZ, SZ)])

    @pl.when(sid == NSUB - 1)
    def _():
        pltpu.sync_copy(zeros_hbm.at[pl.ds(NSUB * SZ, ZTAIL)],
                        acc_sh.at[pl.ds(NSUB * SZ, ZTAIL)])

    pltpu.sync_copy(dsti_hbm.at[cid, sid], dst_v)
    plsc.subcore_barrier()
    base = sid * ET

    def run(src_ref):
        def body(j, carry):
            pltpu.sync_copy(src_ref.at[pl.ds(base + j * CHUNK, CHUNK)], rows0)
            pltpu.sync_copy(rows0, acc_sh.at[dst_v.at[j]], add=True)
            return carry

        lax.fori_loop(0, KC_S, body, 0)
        pltpu.sync_copy(zeros_hbm.at[pl.ds(0, CHUNK - TAIL)],
                        rows0.at[pl.ds(TAIL, CHUNK - TAIL)])
        pltpu.sync_copy(src_ref.at[pl.ds(base + KC_S * CHUNK, TAIL)],
                        rows0.at[pl.ds(0, TAIL)])
        pltpu.sync_copy(rows0, acc_sh.at[dst_v.at[KC_S]], add=True)

    @pl.when(cid == 0)
    def _():
        run(r_hbm)

    @pl.when(cid == 1)
    def _():
        run(ir_hbm)

    plsc.subcore_barrier()
    pltpu.sync_copy(acc_sh.at[pl.ds(sid * SZ, SZ)],
                    s_hbm.at[cid, pl.ds(sid * SZ, SZ)])

    @pl.when(sid == NSUB - 1)
    def _():
        pltpu.sync_copy(acc_sh.at[pl.ds(NSUB * SZ, ZTAIL)],
                        s_hbm.at[cid, pl.ds(NSUB * SZ, ZTAIL)])


_sc_segsum = pl.kernel(
    _sc_segsum_body,
    out_type=jax.ShapeDtypeStruct((NCORE, NACC, D), jnp.float32),
    mesh=plsc.VectorSubcoreMesh(core_axis_name="c", subcore_axis_name="s"),
    scratch_types=[
        pltpu.VMEM((KC_SP, CHUNK), jnp.int32),
        pltpu.VMEM((CHUNK, D), jnp.float32),
        pltpu.VMEM_SHARED((NACC, D), jnp.float32),
        pltpu.SemaphoreType.DMA,
    ],
)


def _wprep_body(wrel_ref, win_ref, wout_ref, ab_ref, c6_ref):
    ii = lax.broadcasted_iota(jnp.int32, (D, D), 0)
    jj = lax.broadcasted_iota(jnp.int32, (D, D), 1)
    c = (ii == jj).astype(jnp.float32)
    for l in range(NL):
        ab_ref[0, l] = jnp.dot(c, win_ref[l], preferred_element_type=jnp.float32)
        ab_ref[1, l] = jnp.dot(c, wout_ref[l], preferred_element_type=jnp.float32)
        c = jnp.dot(c, wrel_ref[l], preferred_element_type=jnp.float32)
    c6_ref[...] = c


_wprep = pl.pallas_call(
    _wprep_body,
    out_shape=(jax.ShapeDtypeStruct((2, NL, D, D), jnp.float32),
               jax.ShapeDtypeStruct((D, D), jnp.float32)),
)


RB = 1000


def _dense_body(g_ref, s_ref, x_ref, w_ref, b_ref, o_ref):
    acc = jnp.dot(g_ref[0, 0], w_ref[0], preferred_element_type=jnp.float32)
    acc += jnp.dot(g_ref[1, 0], w_ref[1], preferred_element_type=jnp.float32)
    acc += jnp.dot(x_ref[...], w_ref[2], preferred_element_type=jnp.float32)
    acc -= jnp.dot(s_ref[0], w_ref[3], preferred_element_type=jnp.float32)
    acc -= jnp.dot(s_ref[1], w_ref[4], preferred_element_type=jnp.float32)
    o_ref[...] = jnp.tanh(acc / 3.0 + b_ref[...])


_dense = pl.pallas_call(
    _dense_body,
    grid=(N // RB,),
    in_specs=[
        pl.BlockSpec((2, 1, RB, D), lambda i: (0, i // (NH // RB),
                                               i % (NH // RB), 0)),
        pl.BlockSpec((2, RB, D), lambda i: (0, i, 0)),
        pl.BlockSpec((RB, D), lambda i: (i, 0)),
        pl.BlockSpec((5, D, D), lambda i: (0, 0, 0)),
        pl.BlockSpec((1, D), lambda i: (0, 0)),
    ],
    out_specs=pl.BlockSpec((RB, D), lambda i: (i, 0)),
    out_shape=jax.ShapeDtypeStruct((N, D), jnp.float32),
)

EB = 3200


def _relmm_body(r_ref, c_ref, o_ref):
    o_ref[...] = jnp.dot(r_ref[...], c_ref[...],
                         preferred_element_type=jnp.float32)


_relmm = pl.pallas_call(
    _relmm_body,
    grid=(E // EB,),
    in_specs=[pl.BlockSpec((EB, D), lambda i: (i, 0)),
              pl.BlockSpec((D, D), lambda i: (0, 0))],
    out_specs=pl.BlockSpec((EB, D), lambda i: (i, 0)),
    out_shape=jax.ShapeDtypeStruct((E, D), jnp.float32),
)


def kernel(node_features, edge_features, inverse_edge_features,
           edge_index, inverse_edge_index, params):
    x = node_features
    r0 = edge_features
    ir0 = inverse_edge_features
    ei = edge_index.astype(jnp.int32)
    iei = inverse_edge_index.astype(jnp.int32)
    src, dst = ei[0], ei[1]
    isrc, idst = iei[0], iei[1]

    hd = NH + (jnp.arange(E, dtype=jnp.int32) % 8)

    def pack_g(a, padval):
        return jnp.pad(a, (0, EPAD - E),
                       constant_values=padval).reshape(NSUB, IR, 128)

    def pack_s(a):
        return jnp.pad(a.reshape(NSUB, ET), ((0, 0), (0, CHUNK - TAIL)),
                       constant_values=DUMMY).reshape(NSUB, KC_SP, CHUNK)

    srcp = jnp.stack([pack_g(src, 0), pack_g(isrc, 0)])
    dst0p = jnp.stack([pack_g(jnp.where(dst < NH, dst, hd), NH),
                       pack_g(jnp.where(idst < NH, idst, hd), NH)])
    dst1p = jnp.stack([pack_g(jnp.where(dst >= NH, dst - NH, hd), NH),
                       pack_g(jnp.where(idst >= NH, idst - NH, hd), NH)])
    dsts = jnp.stack([pack_s(dst), pack_s(idst)])
    zeros = jnp.zeros((NACC, D), jnp.float32)

    wrel = jnp.stack([p['W_rel'] for p in params])
    win = jnp.stack([p['W_in'] for p in params])
    wout = jnp.stack([p['W_out'] for p in params])
    ab, c6 = _wprep(wrel, win, wout)

    s = _sc_segsum(r0, ir0, dsts, zeros)
    x, s = lax.optimization_barrier((x, s))
    for l in range(NL):
        g = _sc_gather(x, srcp, dst0p, dst1p, zeros)
        w5 = jnp.stack([params[l]['W_in'], params[l]['W_out'],
                        params[l]['W_loop'], ab[0, l], ab[1, l]])
        x = _dense(g, s, x, w5, params[l]['b'].reshape(1, D))

    r_out = _relmm(r0, c6)
    ir_out = _relmm(ir0, c6)
    return x, r_out, ir_out


# speedup vs baseline: 1.3283x; 1.3136x over previous
import jax
import jax.numpy as jnp
from jax import lax
from jax.experimental import pallas as pl
from jax.experimental.pallas import tpu as pltpu
from jax.experimental.pallas import tpu_sc as plsc

N = 10000
E = 320000
D = 128
NL = 6
NSUB = 16
NCORE = 2
CHUNK = 128
ET = E // NSUB
KC_S = ET // CHUNK
TAIL = ET - KC_S * CHUNK
KC_SP = KC_S + 1
NACC = 10016
DUMMY = N
SZ = 624
ZTAIL = NACC - NSUB * SZ
XTAIL = N - NSUB * SZ

NH = N // 2
HACC = 5008
HSZ = 304
HTAIL = HACC - NSUB * HSZ
CH = 16
KCG = 1280
IR = KCG * CH // 128
EPAD = NSUB * KCG * CH


def _sc_gather_body(x_hbm, srci_hbm, dst0_hbm, dst1_hbm, zeros_hbm, g_hbm,
                    src_v, dst_v, r0_, r1_, r2_, r3_, x_sh, acc_sh, sem0, sem1):
    rows = (r0_, r1_, r2_, r3_)
    cid = lax.axis_index("c")
    sid = lax.axis_index("s")
    pltpu.sync_copy(x_hbm.at[pl.ds(sid * SZ, SZ)],
                    x_sh.at[pl.ds(sid * SZ, SZ)])

    @pl.when(sid == NSUB - 1)
    def _():
        pltpu.sync_copy(x_hbm.at[pl.ds(NSUB * SZ, XTAIL)],
                        x_sh.at[pl.ds(NSUB * SZ, XTAIL)])

    for h, dsth_hbm in enumerate((dst0_hbm, dst1_hbm)):
        pltpu.sync_copy(zeros_hbm.at[pl.ds(sid * HSZ, HSZ)],
                        acc_sh.at[pl.ds(sid * HSZ, HSZ)])

        @pl.when(sid == NSUB - 1)
        def _():
            pltpu.sync_copy(zeros_hbm.at[pl.ds(NSUB * HSZ, HTAIL)],
                            acc_sh.at[pl.ds(NSUB * HSZ, HTAIL)])

        plsc.subcore_barrier()

        def quad(m, carry):
            @pl.when(lax.rem(m, 16) == 0)
            def _():
                o = lax.div(m, 16) * 8
                pltpu.sync_copy(srci_hbm.at[cid, sid, pl.ds(o, 8)], src_v)
                pltpu.sync_copy(dsth_hbm.at[cid, sid, pl.ds(o, 8)], dst_v)
            cps = []
            sc = []
            for k in range(4):
                c = 4 * m + k
                lr = lax.rem(c, 64) // 8
                of = lax.rem(c, 8) * CH
                si = src_v[lr, pl.ds(of, CH)]
                di = dst_v[lr, pl.ds(of, CH)]
                cps.append((pltpu.async_copy(x_sh.at[si], rows[k], sem0), di))
            for k in range(4):
                cp, di = cps[k]
                cp.wait()
                sc.append(pltpu.async_copy(rows[k], acc_sh.at[di], sem1,
                                           add=True))
            for k in range(4):
                sc[k].wait()
            return carry

        lax.fori_loop(0, KCG // 4, quad, 0)
        plsc.subcore_barrier()
        pltpu.sync_copy(acc_sh.at[pl.ds(sid * HSZ, HSZ)],
                        g_hbm.at[cid, h, pl.ds(sid * HSZ, HSZ)])

        @pl.when(sid == NSUB - 1)
        def _():
            pltpu.sync_copy(acc_sh.at[pl.ds(NSUB * HSZ, HTAIL)],
                            g_hbm.at[cid, h, pl.ds(NSUB * HSZ, HTAIL)])


_sc_gather = pl.kernel(
    _sc_gather_body,
    out_type=jax.ShapeDtypeStruct((NCORE, 2, HACC, D), jnp.float32),
    mesh=plsc.VectorSubcoreMesh(core_axis_name="c", subcore_axis_name="s"),
    scratch_types=[
        pltpu.VMEM((8, 128), jnp.int32),
        pltpu.VMEM((8, 128), jnp.int32),
        pltpu.VMEM((CH, D), jnp.float32),
        pltpu.VMEM((CH, D), jnp.float32),
        pltpu.VMEM((CH, D), jnp.float32),
        pltpu.VMEM((CH, D), jnp.float32),
        pltpu.VMEM_SHARED((N, D), jnp.float32),
        pltpu.VMEM_SHARED((HACC, D), jnp.float32),
        pltpu.SemaphoreType.DMA,
        pltpu.SemaphoreType.DMA,
    ],
)


def _sc_segsum_body(r_hbm, ir_hbm, dsti_hbm, zeros_hbm, s_hbm,
                    dst_v, rows0, acc_sh, sem0):
    cid = lax.axis_index("c")
    sid = lax.axis_index("s")
    pltpu.sync_copy(zeros_hbm.at[pl.ds(sid * SZ, SZ)],
                    acc_sh.at[pl.ds(sid * SZ, SZ)])

    @pl.when(sid == NSUB - 1)
    def _():
        pltpu.sync_copy(zeros_hbm.at[pl.ds(NSUB * SZ, ZTAIL)],
                        acc_sh.at[pl.ds(NSUB * SZ, ZTAIL)])

    pltpu.sync_copy(dsti_hbm.at[cid, sid], dst_v)
    plsc.subcore_barrier()
    base = sid * ET

    def run(src_ref):
        def body(j, carry):
            pltpu.sync_copy(src_ref.at[pl.ds(base + j * CHUNK, CHUNK)], rows0)
            pltpu.sync_copy(rows0, acc_sh.at[dst_v.at[j]], add=True)
            return carry

        lax.fori_loop(0, KC_S, body, 0)
        pltpu.sync_copy(zeros_hbm.at[pl.ds(0, CHUNK - TAIL)],
                        rows0.at[pl.ds(TAIL, CHUNK - TAIL)])
        pltpu.sync_copy(src_ref.at[pl.ds(base + KC_S * CHUNK, TAIL)],
                        rows0.at[pl.ds(0, TAIL)])
        pltpu.sync_copy(rows0, acc_sh.at[dst_v.at[KC_S]], add=True)

    @pl.when(cid == 0)
    def _():
        run(r_hbm)

    @pl.when(cid == 1)
    def _():
        run(ir_hbm)

    plsc.subcore_barrier()
    pltpu.sync_copy(acc_sh.at[pl.ds(sid * SZ, SZ)],
                    s_hbm.at[cid, pl.ds(sid * SZ, SZ)])

    @pl.when(sid == NSUB - 1)
    def _():
        pltpu.sync_copy(acc_sh.at[pl.ds(NSUB * SZ, ZTAIL)],
                        s_hbm.at[cid, pl.ds(NSUB * SZ, ZTAIL)])


_sc_segsum = pl.kernel(
    _sc_segsum_body,
    out_type=jax.ShapeDtypeStruct((NCORE, NACC, D), jnp.float32),
    mesh=plsc.VectorSubcoreMesh(core_axis_name="c", subcore_axis_name="s"),
    scratch_types=[
        pltpu.VMEM((KC_SP, CHUNK), jnp.int32),
        pltpu.VMEM((CHUNK, D), jnp.float32),
        pltpu.VMEM_SHARED((NACC, D), jnp.float32),
        pltpu.SemaphoreType.DMA,
    ],
)


def _wprep_body(wrel_ref, win_ref, wout_ref, ab_ref, c6_ref):
    ii = lax.broadcasted_iota(jnp.int32, (D, D), 0)
    jj = lax.broadcasted_iota(jnp.int32, (D, D), 1)
    c = (ii == jj).astype(jnp.float32)
    for l in range(NL):
        ab_ref[0, l] = jnp.dot(c, win_ref[l], preferred_element_type=jnp.float32)
        ab_ref[1, l] = jnp.dot(c, wout_ref[l], preferred_element_type=jnp.float32)
        c = jnp.dot(c, wrel_ref[l], preferred_element_type=jnp.float32)
    c6_ref[...] = c


_wprep = pl.pallas_call(
    _wprep_body,
    out_shape=(jax.ShapeDtypeStruct((2, NL, D, D), jnp.float32),
               jax.ShapeDtypeStruct((D, D), jnp.float32)),
)


RB = 1000


def _dense_body(g_ref, s_ref, x_ref, w_ref, b_ref, o_ref):
    acc = jnp.dot(g_ref[0, 0], w_ref[0], preferred_element_type=jnp.float32)
    acc += jnp.dot(g_ref[1, 0], w_ref[1], preferred_element_type=jnp.float32)
    acc += jnp.dot(x_ref[...], w_ref[2], preferred_element_type=jnp.float32)
    acc -= jnp.dot(s_ref[0], w_ref[3], preferred_element_type=jnp.float32)
    acc -= jnp.dot(s_ref[1], w_ref[4], preferred_element_type=jnp.float32)
    o_ref[...] = jnp.tanh(acc / 3.0 + b_ref[...])


_dense = pl.pallas_call(
    _dense_body,
    grid=(N // RB,),
    in_specs=[
        pl.BlockSpec((2, 1, RB, D), lambda i: (0, i // (NH // RB),
                                               i % (NH // RB), 0)),
        pl.BlockSpec((2, RB, D), lambda i: (0, i, 0)),
        pl.BlockSpec((RB, D), lambda i: (i, 0)),
        pl.BlockSpec((5, D, D), lambda i: (0, 0, 0)),
        pl.BlockSpec((1, D), lambda i: (0, 0)),
    ],
    out_specs=pl.BlockSpec((RB, D), lambda i: (i, 0)),
    out_shape=jax.ShapeDtypeStruct((N, D), jnp.float32),
)

EB = 3200


def _relmm_body(r_ref, c_ref, o_ref):
    o_ref[...] = jnp.dot(r_ref[...], c_ref[...],
                         preferred_element_type=jnp.float32)


_relmm = pl.pallas_call(
    _relmm_body,
    grid=(E // EB,),
    in_specs=[pl.BlockSpec((EB, D), lambda i: (i, 0)),
              pl.BlockSpec((D, D), lambda i: (0, 0))],
    out_specs=pl.BlockSpec((EB, D), lambda i: (i, 0)),
    out_shape=jax.ShapeDtypeStruct((E, D), jnp.float32),
)


def kernel(node_features, edge_features, inverse_edge_features,
           edge_index, inverse_edge_index, params):
    x = node_features
    r0 = edge_features
    ir0 = inverse_edge_features
    ei = edge_index.astype(jnp.int32)
    iei = inverse_edge_index.astype(jnp.int32)
    src, dst = ei[0], ei[1]
    isrc, idst = iei[0], iei[1]

    hd = NH + (jnp.arange(E, dtype=jnp.int32) % 8)

    def pack_g(a, padval):
        return jnp.pad(a, (0, EPAD - E),
                       constant_values=padval).reshape(NSUB, IR, 128)

    def pack_s(a):
        return jnp.pad(a.reshape(NSUB, ET), ((0, 0), (0, CHUNK - TAIL)),
                       constant_values=DUMMY).reshape(NSUB, KC_SP, CHUNK)

    srcp = jnp.stack([pack_g(src, 0), pack_g(isrc, 0)])
    dst0p = jnp.stack([pack_g(jnp.where(dst < NH, dst, hd), NH),
                       pack_g(jnp.where(idst < NH, idst, hd), NH)])
    dst1p = jnp.stack([pack_g(jnp.where(dst >= NH, dst - NH, hd), NH),
                       pack_g(jnp.where(idst >= NH, idst - NH, hd), NH)])
    dsts = jnp.stack([pack_s(dst), pack_s(idst)])
    zeros = jnp.zeros((NACC, D), jnp.float32)

    wrel = jnp.stack([p['W_rel'] for p in params])
    win = jnp.stack([p['W_in'] for p in params])
    wout = jnp.stack([p['W_out'] for p in params])
    ab, c6 = _wprep(wrel, win, wout)

    s = _sc_segsum(r0, ir0, dsts, zeros)
    x, s = lax.optimization_barrier((x, s))
    for l in range(NL):
        g = _sc_gather(x, srcp, dst0p, dst1p, zeros)
        w5 = jnp.stack([params[l]['W_in'], params[l]['W_out'],
                        params[l]['W_loop'], ab[0, l], ab[1, l]])
        x = _dense(g, s, x, w5, params[l]['b'].reshape(1, D))

    r_out = _relmm(r0, c6)
    ir_out = _relmm(ir0, c6)
    return x, r_out, ir_out
